# Initial kernel scaffold; baseline (speedup 1.0000x reference)
import jax
import jax.numpy as jnp
from jax.experimental import pallas as pl


def kernel(node_features, edge_index, edge_type, bases, comp, root, bias1, w_rel, w_root, bias2):
    raise NotImplementedError("write your pallas kernel here")


# speedup vs baseline: 9.1352x; 9.1352x over previous
import functools

import jax
import jax.numpy as jnp
from jax import lax
from jax.experimental import pallas as pl
from jax.experimental.pallas import tpu as pltpu, tpu_sc as plsc

N_NODES = 10000
N_EDGES = 320000
D = 128
R = 8
NC, NS, L = 2, 16, 16
NW = NC * NS
CHUNK = 128
NCHUNK = N_EDGES // CHUNK
ITERS = (NCHUNK + NW - 1) // NW
NPAD = 10240
ROWS_PER_TILE = NPAD // NS
NR = NPAD * R

_mesh = plsc.VectorSubcoreMesh(core_axis_name="c", subcore_axis_name="s",
                               num_cores=NC, num_subcores=NS)
_sc_params = pltpu.CompilerParams(needs_layout_passes=False)


def _wid(c, s):
    return s * NC + c


@functools.partial(
    pl.kernel,
    out_type=jax.ShapeDtypeStruct((NW, NR), jnp.float32),
    mesh=_mesh,
    compiler_params=_sc_params,
    scratch_types=[
        pltpu.VMEM((NR,), jnp.float32),
        pltpu.VMEM((CHUNK,), jnp.int32),
        pltpu.VMEM((CHUNK,), jnp.int32),
    ],
)
def _sc_counts(dst_hbm, type_hbm, cnt_hbm, acc, dbuf, tbuf):
    c = lax.axis_index("c")
    s = lax.axis_index("s")
    wid = _wid(c, s)

    def _z(i, _):
        acc[pl.ds(i * 16, 16)] = jnp.zeros((16,), jnp.float32)
        return 0
    lax.fori_loop(0, NR // 16, _z, 0)

    ones = jnp.ones((16,), jnp.float32)

    def _step(i, _):
        cid = wid + i * NW

        @pl.when(cid < NCHUNK)
        def _():
            base = cid * CHUNK
            pltpu.sync_copy(dst_hbm.at[pl.ds(base, CHUNK)], dbuf)
            pltpu.sync_copy(type_hbm.at[pl.ds(base, CHUNK)], tbuf)
            for j in range(CHUNK // 16):
                dv = dbuf[pl.ds(j * 16, 16)]
                tv = tbuf[pl.ds(j * 16, 16)]
                plsc.addupdate_scatter(acc, [dv * 8 + tv], ones)
        return 0

    lax.fori_loop(0, ITERS, _step, 0)
    pltpu.sync_copy(acc, cnt_hbm.at[wid])


@functools.partial(
    pl.kernel,
    out_type=jax.ShapeDtypeStruct((N_EDGES,), jnp.float32),
    mesh=_mesh,
    compiler_params=_sc_params,
    scratch_types=[
        pltpu.VMEM((NR,), jnp.float32),
        pltpu.VMEM((CHUNK,), jnp.int32),
        pltpu.VMEM((CHUNK,), jnp.int32),
        pltpu.VMEM((CHUNK,), jnp.float32),
    ],
)
def _sc_edge_w(dst_hbm, type_hbm, inv_hbm, w_hbm, inv_v, dbuf, tbuf, wbuf):
    c = lax.axis_index("c")
    s = lax.axis_index("s")
    wid = _wid(c, s)
    pltpu.sync_copy(inv_hbm, inv_v)

    def _step(i, _):
        cid = wid + i * NW

        @pl.when(cid < NCHUNK)
        def _():
            base = cid * CHUNK
            pltpu.sync_copy(dst_hbm.at[pl.ds(base, CHUNK)], dbuf)
            pltpu.sync_copy(type_hbm.at[pl.ds(base, CHUNK)], tbuf)
            for j in range(CHUNK // 16):
                dv = dbuf[pl.ds(j * 16, 16)]
                tv = tbuf[pl.ds(j * 16, 16)]
                wbuf[pl.ds(j * 16, 16)] = plsc.load_gather(inv_v, [dv * 8 + tv])
            pltpu.sync_copy(wbuf, w_hbm.at[pl.ds(base, CHUNK)])
        return 0

    lax.fori_loop(0, ITERS, _step, 0)


@functools.partial(
    pl.kernel,
    out_type=jax.ShapeDtypeStruct((NC, NPAD, D), jnp.float32),
    mesh=_mesh,
    compiler_params=_sc_params,
    scratch_types=[
        pltpu.VMEM((CHUNK, D), jnp.float32),
        pltpu.VMEM((CHUNK,), jnp.int32),
        pltpu.VMEM((CHUNK,), jnp.int32),
        pltpu.VMEM((CHUNK,), jnp.int32),
        pltpu.VMEM((CHUNK,), jnp.int32),
        pltpu.VMEM((CHUNK,), jnp.float32),
        pltpu.VMEM_SHARED((NPAD, D), jnp.float32),
        pltpu.SemaphoreType.DMA,
    ],
)
def _sc_main(src_hbm, dst_hbm, type_hbm, y_hbm, w_hbm, agg_hbm,
             msg, sbuf, dbuf, tbuf, gidx, wbuf, agg_sh, sem1):
    c = lax.axis_index("c")
    s = lax.axis_index("s")
    wid = _wid(c, s)
    row0 = s * ROWS_PER_TILE

    def _zm(i, _):
        for f in range(D // 16):
            msg[i, pl.ds(f * 16, 16)] = jnp.zeros((16,), jnp.float32)
        return 0
    lax.fori_loop(0, CHUNK, _zm, 0)
    for k in range(ROWS_PER_TILE // CHUNK):
        pltpu.sync_copy(msg, agg_sh.at[pl.ds(row0 + k * CHUNK, CHUNK)])
    plsc.subcore_barrier()

    def _step(i, _):
        cid = wid + i * NW

        @pl.when(cid < NCHUNK)
        def _():
            base = cid * CHUNK
            pltpu.sync_copy(src_hbm.at[pl.ds(base, CHUNK)], sbuf)
            pltpu.sync_copy(dst_hbm.at[pl.ds(base, CHUNK)], dbuf)
            pltpu.sync_copy(type_hbm.at[pl.ds(base, CHUNK)], tbuf)
            pltpu.sync_copy(w_hbm.at[pl.ds(base, CHUNK)], wbuf)
            for j in range(CHUNK // 16):
                sv = sbuf[pl.ds(j * 16, 16)]
                tv = tbuf[pl.ds(j * 16, 16)]
                gidx[pl.ds(j * 16, 16)] = sv * 8 + tv
            pltpu.async_copy(y_hbm.at[gidx], msg, sem1).wait()

            def _scale(e, _):
                w = plsc.load_gather(wbuf, [jnp.full((16,), e, jnp.int32)])
                for f in range(D // 16):
                    msg[e, pl.ds(f * 16, 16)] = msg[e, pl.ds(f * 16, 16)] * w
                return 0
            lax.fori_loop(0, CHUNK, _scale, 0)
            pltpu.sync_copy(msg, agg_sh.at[dbuf], add=True)
        return 0

    lax.fori_loop(0, ITERS, _step, 0)
    plsc.subcore_barrier()
    pltpu.sync_copy(agg_sh.at[pl.ds(row0, ROWS_PER_TILE)],
                    agg_hbm.at[c, pl.ds(row0, ROWS_PER_TILE)])


@functools.partial(
    pl.kernel,
    out_type=jax.ShapeDtypeStruct((NC, NPAD, D), jnp.float32),
    mesh=_mesh,
    compiler_params=_sc_params,
    scratch_types=[
        pltpu.VMEM((CHUNK, D), jnp.float32),
        pltpu.VMEM((CHUNK,), jnp.int32),
        pltpu.VMEM((CHUNK,), jnp.int32),
        pltpu.VMEM_SHARED((NPAD, D), jnp.float32),
        pltpu.SemaphoreType.DMA,
    ],
)
def _sc_neigh(src_hbm, dst_hbm, x1_hbm, out_hbm, msg, sbuf, dbuf, agg_sh, sem):
    c = lax.axis_index("c")
    s = lax.axis_index("s")
    wid = _wid(c, s)
    row0 = s * ROWS_PER_TILE

    def _zm(i, _):
        for f in range(D // 16):
            msg[i, pl.ds(f * 16, 16)] = jnp.zeros((16,), jnp.float32)
        return 0
    lax.fori_loop(0, CHUNK, _zm, 0)
    for k in range(ROWS_PER_TILE // CHUNK):
        pltpu.sync_copy(msg, agg_sh.at[pl.ds(row0 + k * CHUNK, CHUNK)])
    plsc.subcore_barrier()

    def _step(i, _):
        cid = wid + i * NW

        @pl.when(cid < NCHUNK)
        def _():
            base = cid * CHUNK
            pltpu.sync_copy(src_hbm.at[pl.ds(base, CHUNK)], sbuf)
            pltpu.sync_copy(dst_hbm.at[pl.ds(base, CHUNK)], dbuf)
            pltpu.async_copy(x1_hbm.at[sbuf], msg, sem).wait()
            pltpu.sync_copy(msg, agg_sh.at[dbuf], add=True)
        return 0

    lax.fori_loop(0, ITERS, _step, 0)
    plsc.subcore_barrier()
    pltpu.sync_copy(agg_sh.at[pl.ds(row0, ROWS_PER_TILE)],
                    out_hbm.at[c, pl.ds(row0, ROWS_PER_TILE)])


_BN = 512
_BF = _BN * R


def _tc_a_body(x_ref, bases_ref, comp_ref, root_ref, b1_ref, cnt_ref,
               y_ref, xroot_ref, inv_ref):
    xb = x_ref[...]
    bases = bases_ref[...].reshape(10, D * D)
    wcat = jnp.dot(comp_ref[...], bases,
                   preferred_element_type=jnp.float32)
    for r in range(R):
        wr = wcat[r].reshape(D, D)
        y_ref[:, r, :] = jnp.dot(xb, wr, preferred_element_type=jnp.float32)
    xroot_ref[...] = jnp.dot(xb, root_ref[...],
                             preferred_element_type=jnp.float32) + b1_ref[...]
    cnt = jnp.sum(cnt_ref[...], axis=0)
    inv_ref[...] = 1.0 / jnp.maximum(cnt, 1.0)


def _tc_a(x, bases, comp, root, bias1, cnt):
    grid = (NPAD // _BN,)
    return pl.pallas_call(
        _tc_a_body,
        grid=grid,
        in_specs=[
            pl.BlockSpec((_BN, D), lambda i: (i, 0)),
            pl.BlockSpec((10, D, D), lambda i: (0, 0, 0)),
            pl.BlockSpec((R, 10), lambda i: (0, 0)),
            pl.BlockSpec((D, D), lambda i: (0, 0)),
            pl.BlockSpec((1, D), lambda i: (0, 0)),
            pl.BlockSpec((NW, _BN, R), lambda i: (0, i, 0)),
        ],
        out_specs=[
            pl.BlockSpec((_BN, R, D), lambda i: (i, 0, 0)),
            pl.BlockSpec((_BN, D), lambda i: (i, 0)),
            pl.BlockSpec((_BN, R), lambda i: (i, 0)),
        ],
        out_shape=[
            jax.ShapeDtypeStruct((NPAD, R, D), jnp.float32),
            jax.ShapeDtypeStruct((NPAD, D), jnp.float32),
            jax.ShapeDtypeStruct((NPAD, R), jnp.float32),
        ],
    )(x, bases, comp, root, bias1, cnt)


def _tc_b_body(xroot_ref, agg_ref, wroot_ref, b2_ref, x1_ref, xw_ref):
    x1 = xroot_ref[...] + agg_ref[0] + agg_ref[1]
    x1_ref[...] = x1
    xw_ref[...] = jnp.dot(x1, wroot_ref[...],
                          preferred_element_type=jnp.float32) + b2_ref[...]


def _tc_b(xroot, agg, w_root, bias2):
    grid = (NPAD // _BN,)
    return pl.pallas_call(
        _tc_b_body,
        grid=grid,
        in_specs=[
            pl.BlockSpec((_BN, D), lambda i: (i, 0)),
            pl.BlockSpec((NC, _BN, D), lambda i: (0, i, 0)),
            pl.BlockSpec((D, D), lambda i: (0, 0)),
            pl.BlockSpec((1, D), lambda i: (0, 0)),
        ],
        out_specs=[
            pl.BlockSpec((_BN, D), lambda i: (i, 0)),
            pl.BlockSpec((_BN, D), lambda i: (i, 0)),
        ],
        out_shape=[
            jax.ShapeDtypeStruct((NPAD, D), jnp.float32),
            jax.ShapeDtypeStruct((NPAD, D), jnp.float32),
        ],
    )(xroot, agg, w_root, bias2)


def _tc_c_body(xw_ref, neigh_ref, wrel_ref, out_ref):
    neigh = neigh_ref[0] + neigh_ref[1]
    out_ref[...] = xw_ref[...] + jnp.dot(neigh, wrel_ref[...],
                                         preferred_element_type=jnp.float32)


def _tc_c(xw, neigh, w_rel):
    grid = (NPAD // _BN,)
    return pl.pallas_call(
        _tc_c_body,
        grid=grid,
        in_specs=[
            pl.BlockSpec((_BN, D), lambda i: (i, 0)),
            pl.BlockSpec((NC, _BN, D), lambda i: (0, i, 0)),
            pl.BlockSpec((D, D), lambda i: (0, 0)),
        ],
        out_specs=pl.BlockSpec((_BN, D), lambda i: (i, 0)),
        out_shape=jax.ShapeDtypeStruct((NPAD, D), jnp.float32),
    )(xw, neigh, w_rel)


def kernel(node_features, edge_index, edge_type, bases, comp, root, bias1,
           w_rel, w_root, bias2):
    src = edge_index[0].astype(jnp.int32)
    dst = edge_index[1].astype(jnp.int32)
    et = edge_type.astype(jnp.int32)
    xpad = jnp.pad(node_features, ((0, NPAD - N_NODES), (0, 0)))

    cnt = _sc_counts(dst, et).reshape(NW, NPAD, R)
    y, xroot, inv = _tc_a(xpad, bases, comp, root, bias1.reshape(1, D), cnt)
    w = _sc_edge_w(dst, et, inv.reshape(NR))
    agg = _sc_main(src, dst, et, y.reshape(NR, D), w)
    x1, xw = _tc_b(xroot, agg, w_root, bias2.reshape(1, D))
    neigh = _sc_neigh(src, dst, x1)
    out = _tc_c(xw, neigh, w_rel)
    return out[:N_NODES]


# speedup vs baseline: 11.3119x; 1.2383x over previous
import functools

import jax
import jax.numpy as jnp
from jax import lax
from jax.experimental import pallas as pl
from jax.experimental.pallas import tpu as pltpu, tpu_sc as plsc

N_NODES = 10000
N_EDGES = 320000
D = 128
R = 8
NC, NS, L = 2, 16, 16
NW = NC * NS
CHUNK = 128
NCHUNK = N_EDGES // CHUNK
ITERS = (NCHUNK + NW - 1) // NW
NPAD = 10240
ROWS_PER_TILE = NPAD // NS
NR = NPAD * R

_mesh = plsc.VectorSubcoreMesh(core_axis_name="c", subcore_axis_name="s",
                               num_cores=NC, num_subcores=NS)
_sc_params = pltpu.CompilerParams(needs_layout_passes=False)


def _wid(c, s):
    return s * NC + c


@functools.partial(
    pl.kernel,
    out_type=jax.ShapeDtypeStruct((NW, NR), jnp.float32),
    mesh=_mesh,
    compiler_params=_sc_params,
    scratch_types=[
        pltpu.VMEM((NR,), jnp.float32),
        pltpu.VMEM((CHUNK,), jnp.int32),
        pltpu.VMEM((CHUNK,), jnp.int32),
        pltpu.SemaphoreType.DMA,
    ],
)
def _sc_counts(dst_hbm, type_hbm, cnt_hbm, acc, dbuf, tbuf, sem):
    c = lax.axis_index("c")
    s = lax.axis_index("s")
    wid = _wid(c, s)

    def _z(i, _):
        acc[pl.ds(i * 16, 16)] = jnp.zeros((16,), jnp.float32)
        return 0
    lax.fori_loop(0, NR // 16, _z, 0)

    ones = jnp.ones((16,), jnp.float32)

    def _step(i, _):
        cid = wid + i * NW

        @pl.when(cid < NCHUNK)
        def _():
            base = cid * CHUNK
            cp1 = pltpu.async_copy(dst_hbm.at[pl.ds(base, CHUNK)], dbuf, sem)
            cp2 = pltpu.async_copy(type_hbm.at[pl.ds(base, CHUNK)], tbuf, sem)
            cp1.wait()
            cp2.wait()
            for j in range(CHUNK // 16):
                dv = dbuf[pl.ds(j * 16, 16)]
                tv = tbuf[pl.ds(j * 16, 16)]
                plsc.addupdate_scatter(acc, [dv * 8 + tv], ones)
        return 0

    lax.fori_loop(0, ITERS, _step, 0)
    pltpu.sync_copy(acc, cnt_hbm.at[wid])


@functools.partial(
    pl.kernel,
    out_type=jax.ShapeDtypeStruct((N_EDGES,), jnp.float32),
    mesh=_mesh,
    compiler_params=_sc_params,
    scratch_types=[
        pltpu.VMEM((NR,), jnp.float32),
        pltpu.VMEM((CHUNK,), jnp.int32),
        pltpu.VMEM((CHUNK,), jnp.int32),
        pltpu.VMEM((CHUNK,), jnp.float32),
        pltpu.SemaphoreType.DMA,
    ],
)
def _sc_edge_w(dst_hbm, type_hbm, inv_hbm, w_hbm, inv_v, dbuf, tbuf, wbuf, sem):
    c = lax.axis_index("c")
    s = lax.axis_index("s")
    wid = _wid(c, s)
    pltpu.sync_copy(inv_hbm, inv_v)

    def _step(i, _):
        cid = wid + i * NW

        @pl.when(cid < NCHUNK)
        def _():
            base = cid * CHUNK
            cp1 = pltpu.async_copy(dst_hbm.at[pl.ds(base, CHUNK)], dbuf, sem)
            cp2 = pltpu.async_copy(type_hbm.at[pl.ds(base, CHUNK)], tbuf, sem)
            cp1.wait()
            cp2.wait()
            for j in range(CHUNK // 16):
                dv = dbuf[pl.ds(j * 16, 16)]
                tv = tbuf[pl.ds(j * 16, 16)]
                wbuf[pl.ds(j * 16, 16)] = plsc.load_gather(inv_v, [dv * 8 + tv])
            pltpu.sync_copy(wbuf, w_hbm.at[pl.ds(base, CHUNK)])
        return 0

    lax.fori_loop(0, ITERS, _step, 0)


@functools.partial(
    pl.kernel,
    out_type=jax.ShapeDtypeStruct((NC, NPAD, D), jnp.float32),
    mesh=_mesh,
    compiler_params=_sc_params,
    scratch_types=[
        pltpu.VMEM((CHUNK, D), jnp.float32),
        pltpu.VMEM((CHUNK,), jnp.int32),
        pltpu.VMEM((CHUNK,), jnp.int32),
        pltpu.VMEM((CHUNK,), jnp.int32),
        pltpu.VMEM((CHUNK,), jnp.int32),
        pltpu.VMEM((CHUNK,), jnp.float32),
        pltpu.VMEM_SHARED((NPAD, D), jnp.float32),
        pltpu.SemaphoreType.DMA,
    ],
)
def _sc_main(src_hbm, dst_hbm, type_hbm, y_hbm, w_hbm, agg_hbm,
             msg, sbuf, dbuf, tbuf, gidx, wbuf, agg_sh, sem1):
    c = lax.axis_index("c")
    s = lax.axis_index("s")
    wid = _wid(c, s)
    row0 = s * ROWS_PER_TILE

    def _zm(i, _):
        for f in range(D // 16):
            msg[i, pl.ds(f * 16, 16)] = jnp.zeros((16,), jnp.float32)
        return 0
    lax.fori_loop(0, CHUNK, _zm, 0)
    for k in range(ROWS_PER_TILE // CHUNK):
        pltpu.sync_copy(msg, agg_sh.at[pl.ds(row0 + k * CHUNK, CHUNK)])
    plsc.subcore_barrier()

    def _step(i, _):
        cid = wid + i * NW

        @pl.when(cid < NCHUNK)
        def _():
            base = cid * CHUNK
            cps = [pltpu.async_copy(src_hbm.at[pl.ds(base, CHUNK)], sbuf, sem1),
                   pltpu.async_copy(dst_hbm.at[pl.ds(base, CHUNK)], dbuf, sem1),
                   pltpu.async_copy(type_hbm.at[pl.ds(base, CHUNK)], tbuf, sem1),
                   pltpu.async_copy(w_hbm.at[pl.ds(base, CHUNK)], wbuf, sem1)]
            for cp in cps:
                cp.wait()
            for j in range(CHUNK // 16):
                sv = sbuf[pl.ds(j * 16, 16)]
                tv = tbuf[pl.ds(j * 16, 16)]
                gidx[pl.ds(j * 16, 16)] = sv * 8 + tv
            pltpu.async_copy(y_hbm.at[gidx], msg, sem1).wait()

            def _scale(e, _):
                w = plsc.load_gather(wbuf, [jnp.full((16,), e, jnp.int32)])
                for f in range(D // 16):
                    msg[e, pl.ds(f * 16, 16)] = msg[e, pl.ds(f * 16, 16)] * w
                return 0
            lax.fori_loop(0, CHUNK, _scale, 0, unroll=8)
            pltpu.sync_copy(msg, agg_sh.at[dbuf], add=True)
        return 0

    lax.fori_loop(0, ITERS, _step, 0)
    plsc.subcore_barrier()
    pltpu.sync_copy(agg_sh.at[pl.ds(row0, ROWS_PER_TILE)],
                    agg_hbm.at[c, pl.ds(row0, ROWS_PER_TILE)])


@functools.partial(
    pl.kernel,
    out_type=jax.ShapeDtypeStruct((NC, NPAD, D), jnp.float32),
    mesh=_mesh,
    compiler_params=_sc_params,
    scratch_types=[
        pltpu.VMEM((CHUNK, D), jnp.float32),
        pltpu.VMEM((CHUNK,), jnp.int32),
        pltpu.VMEM((CHUNK,), jnp.int32),
        pltpu.VMEM_SHARED((NPAD, D), jnp.float32),
        pltpu.SemaphoreType.DMA,
    ],
)
def _sc_neigh(src_hbm, dst_hbm, x1_hbm, out_hbm, msg, sbuf, dbuf, agg_sh, sem):
    c = lax.axis_index("c")
    s = lax.axis_index("s")
    wid = _wid(c, s)
    row0 = s * ROWS_PER_TILE

    def _zm(i, _):
        for f in range(D // 16):
            msg[i, pl.ds(f * 16, 16)] = jnp.zeros((16,), jnp.float32)
        return 0
    lax.fori_loop(0, CHUNK, _zm, 0)
    for k in range(ROWS_PER_TILE // CHUNK):
        pltpu.sync_copy(msg, agg_sh.at[pl.ds(row0 + k * CHUNK, CHUNK)])
    plsc.subcore_barrier()

    def _step(i, _):
        cid = wid + i * NW

        @pl.when(cid < NCHUNK)
        def _():
            base = cid * CHUNK
            cp1 = pltpu.async_copy(src_hbm.at[pl.ds(base, CHUNK)], sbuf, sem)
            cp2 = pltpu.async_copy(dst_hbm.at[pl.ds(base, CHUNK)], dbuf, sem)
            cp1.wait()
            cp2.wait()
            pltpu.async_copy(x1_hbm.at[sbuf], msg, sem).wait()
            pltpu.sync_copy(msg, agg_sh.at[dbuf], add=True)
        return 0

    lax.fori_loop(0, ITERS, _step, 0)
    plsc.subcore_barrier()
    pltpu.sync_copy(agg_sh.at[pl.ds(row0, ROWS_PER_TILE)],
                    out_hbm.at[c, pl.ds(row0, ROWS_PER_TILE)])


_BN = 512
_BF = _BN * R


def _tc_a_body(x_ref, bases_ref, comp_ref, root_ref, b1_ref, cnt_ref,
               y_ref, xroot_ref, inv_ref):
    xb = x_ref[...]
    bases = bases_ref[...].reshape(10, D * D)
    wcat = jnp.dot(comp_ref[...], bases,
                   preferred_element_type=jnp.float32)
    for r in range(R):
        wr = wcat[r].reshape(D, D)
        y_ref[:, r, :] = jnp.dot(xb, wr, preferred_element_type=jnp.float32)
    xroot_ref[...] = jnp.dot(xb, root_ref[...],
                             preferred_element_type=jnp.float32) + b1_ref[...]
    cnt = jnp.sum(cnt_ref[...], axis=0)
    inv_ref[...] = 1.0 / jnp.maximum(cnt, 1.0)


def _tc_a(x, bases, comp, root, bias1, cnt):
    grid = (NPAD // _BN,)
    return pl.pallas_call(
        _tc_a_body,
        grid=grid,
        in_specs=[
            pl.BlockSpec((_BN, D), lambda i: (i, 0)),
            pl.BlockSpec((10, D, D), lambda i: (0, 0, 0)),
            pl.BlockSpec((R, 10), lambda i: (0, 0)),
            pl.BlockSpec((D, D), lambda i: (0, 0)),
            pl.BlockSpec((1, D), lambda i: (0, 0)),
            pl.BlockSpec((NW, _BN, R), lambda i: (0, i, 0)),
        ],
        out_specs=[
            pl.BlockSpec((_BN, R, D), lambda i: (i, 0, 0)),
            pl.BlockSpec((_BN, D), lambda i: (i, 0)),
            pl.BlockSpec((_BN, R), lambda i: (i, 0)),
        ],
        out_shape=[
            jax.ShapeDtypeStruct((NPAD, R, D), jnp.float32),
            jax.ShapeDtypeStruct((NPAD, D), jnp.float32),
            jax.ShapeDtypeStruct((NPAD, R), jnp.float32),
        ],
    )(x, bases, comp, root, bias1, cnt)


def _tc_b_body(xroot_ref, agg_ref, wroot_ref, b2_ref, x1_ref, xw_ref):
    x1 = xroot_ref[...] + agg_ref[0] + agg_ref[1]
    x1_ref[...] = x1
    xw_ref[...] = jnp.dot(x1, wroot_ref[...],
                          preferred_element_type=jnp.float32) + b2_ref[...]


def _tc_b(xroot, agg, w_root, bias2):
    grid = (NPAD // _BN,)
    return pl.pallas_call(
        _tc_b_body,
        grid=grid,
        in_specs=[
            pl.BlockSpec((_BN, D), lambda i: (i, 0)),
            pl.BlockSpec((NC, _BN, D), lambda i: (0, i, 0)),
            pl.BlockSpec((D, D), lambda i: (0, 0)),
            pl.BlockSpec((1, D), lambda i: (0, 0)),
        ],
        out_specs=[
            pl.BlockSpec((_BN, D), lambda i: (i, 0)),
            pl.BlockSpec((_BN, D), lambda i: (i, 0)),
        ],
        out_shape=[
            jax.ShapeDtypeStruct((NPAD, D), jnp.float32),
            jax.ShapeDtypeStruct((NPAD, D), jnp.float32),
        ],
    )(xroot, agg, w_root, bias2)


def _tc_c_body(xw_ref, neigh_ref, wrel_ref, out_ref):
    neigh = neigh_ref[0] + neigh_ref[1]
    out_ref[...] = xw_ref[...] + jnp.dot(neigh, wrel_ref[...],
                                         preferred_element_type=jnp.float32)


def _tc_c(xw, neigh, w_rel):
    grid = (NPAD // _BN,)
    return pl.pallas_call(
        _tc_c_body,
        grid=grid,
        in_specs=[
            pl.BlockSpec((_BN, D), lambda i: (i, 0)),
            pl.BlockSpec((NC, _BN, D), lambda i: (0, i, 0)),
            pl.BlockSpec((D, D), lambda i: (0, 0)),
        ],
        out_specs=pl.BlockSpec((_BN, D), lambda i: (i, 0)),
        out_shape=jax.ShapeDtypeStruct((NPAD, D), jnp.float32),
    )(xw, neigh, w_rel)


def kernel(node_features, edge_index, edge_type, bases, comp, root, bias1,
           w_rel, w_root, bias2):
    src = edge_index[0].astype(jnp.int32)
    dst = edge_index[1].astype(jnp.int32)
    et = edge_type.astype(jnp.int32)
    xpad = jnp.pad(node_features, ((0, NPAD - N_NODES), (0, 0)))

    cnt = _sc_counts(dst, et).reshape(NW, NPAD, R)
    y, xroot, inv = _tc_a(xpad, bases, comp, root, bias1.reshape(1, D), cnt)
    w = _sc_edge_w(dst, et, inv.reshape(NR))
    agg = _sc_main(src, dst, et, y.reshape(NR, D), w)
    x1, xw = _tc_b(xroot, agg, w_root, bias2.reshape(1, D))
    neigh = _sc_neigh(src, dst, x1)
    out = _tc_c(xw, neigh, w_rel)
    return out[:N_NODES]
